# baseline (speedup 1.0000x reference)
import math

import jax
import jax.numpy as jnp
from jax import lax
from jax.experimental import pallas as pl
from jax.experimental.pallas import tpu as pltpu
from jax.experimental.pallas import tpu_sc as plsc

D_MODEL = 64
SCALE = math.sqrt(D_MODEL)
LANES = 16
B, S = 4096, 200
NBJ = B // 128
NSI = S // 8
CS = 2
CHUNK = CS * 128

_info = plsc.get_sparse_core_info()
NC, NS = _info.num_cores, _info.num_subcores


def _emb_body(table_hbm, x4_hbm, out_hbm,
              i0, i1, p0, p1, q0, q1, g0, g1, d0, d1,
              gsem0, gsem1, wsem0, wsem1):
    bj = lax.axis_index("s") * NC + lax.axis_index("c")
    ibuf = i0
    del i1
    pbuf, qbuf = (p0, p1), (q0, q1)
    gbuf, dbuf = (g0, g1), (d0, d1)
    gsem, wsem = (gsem0, gsem1), (wsem0, wsem1)

    lane = lax.iota(jnp.int32, LANES)
    rowsel = [j * LANES + lane for j in range(CHUNK // LANES)]

    def prep_chunk(s, b):
        @pl.when((s & 7) == 0)
        def _():
            pltpu.sync_copy(x4_hbm.at[s >> 3, bj], ibuf)

        for k in range(CHUNK // LANES):
            v = ibuf[(s & 7) + (k >> 3), pl.ds((k & 7) * LANES, LANES)]
            pbuf[b][pl.ds(k * LANES, LANES)] = v >> 1
            qbuf[b][pl.ds(k * LANES, LANES)] = (v & 1) << 6

    def start_gather(b):
        pltpu.async_copy(table_hbm.at[pbuf[b]], gbuf[b], gsem[b])

    def wait_gather(b):
        pltpu.make_async_copy(table_hbm.at[pbuf[b]], gbuf[b], gsem[b]).wait()

    def start_write(s, b):
        pltpu.async_copy(dbuf[b], out_hbm.at[pl.ds(s, CS), :, bj], wsem[b])

    def wait_write(b):
        pltpu.make_async_copy(
            dbuf[b], out_hbm.at[pl.ds(0, CS), :, bj], wsem[b]).wait()

    prep_chunk(0, 0)
    start_gather(0)
    prep_chunk(CS, 1)
    start_gather(1)

    def do_pair(step, carry):
        for b in (0, 1):
            s = (step * 2 + b) * CS
            wait_gather(b)

            @pl.when(s >= 2 * CS)
            def _():
                wait_write(b)

            for sl in range(CS):
                for j in range(128 // LANES):
                    par = qbuf[b][pl.ds((sl * 128 + j * LANES), LANES)]
                    rows = rowsel[sl * 8 + j]

                    def trans_d(dd, c):
                        vals = plsc.load_gather(gbuf[b], [rows, par + dd])
                        dbuf[b][sl, dd >> 3,
                                dd & 7, pl.ds(j * LANES, LANES)] = vals * SCALE
                        return c

                    lax.fori_loop(0, D_MODEL, trans_d, 0, unroll=8)

            start_write(s, b)

            @pl.when(s + 2 * CS < S)
            def _():
                prep_chunk(s + 2 * CS, b)
                start_gather(b)
        return carry

    lax.fori_loop(0, S // (2 * CS), do_pair, 0)
    wait_write(0)
    wait_write(1)


def kernel(x, lut):
    x4 = x.astype(jnp.int32).reshape(NBJ, 128, NSI, 8).transpose(2, 0, 3, 1)
    lutp = lut.reshape(VOCAB_PAIRS, 128)

    out5 = pl.kernel(
        _emb_body,
        out_type=jax.ShapeDtypeStruct((S, 8, NBJ, 8, 128), jnp.float32),
        mesh=plsc.VectorSubcoreMesh(core_axis_name="c", subcore_axis_name="s"),
        compiler_params=pltpu.CompilerParams(needs_layout_passes=False),
        scratch_types=[
            pltpu.VMEM((8, 128), jnp.int32),
            pltpu.VMEM((8, 128), jnp.int32),
            pltpu.VMEM((CHUNK,), jnp.int32),
            pltpu.VMEM((CHUNK,), jnp.int32),
            pltpu.VMEM((CHUNK,), jnp.int32),
            pltpu.VMEM((CHUNK,), jnp.int32),
            pltpu.VMEM((CHUNK, 128), jnp.float32),
            pltpu.VMEM((CHUNK, 128), jnp.float32),
            pltpu.VMEM((CS, 8, 8, 128), jnp.float32),
            pltpu.VMEM((CS, 8, 8, 128), jnp.float32),
            pltpu.SemaphoreType.DMA,
            pltpu.SemaphoreType.DMA,
            pltpu.SemaphoreType.DMA,
            pltpu.SemaphoreType.DMA,
        ],
    )(lutp, x4)

    o = (out5.transpose(2, 4, 0, 1, 3)
         .reshape(B, S, D_MODEL))
    return o


VOCAB_PAIRS = 500000


# speedup vs baseline: 1.7820x; 1.7820x over previous
import math

import jax
import jax.numpy as jnp
from jax import lax
from jax.experimental import pallas as pl
from jax.experimental.pallas import tpu as pltpu
from jax.experimental.pallas import tpu_sc as plsc

D_MODEL = 64
SCALE = math.sqrt(D_MODEL)
LANES = 16
B, S = 4096, 200
NBJ = B // 128
NSI = S // 8
CS = 2
CHUNK = CS * 128
PITCH = 129

_info = plsc.get_sparse_core_info()
NC, NS = _info.num_cores, _info.num_subcores


def _emb_body(table_hbm, x4_hbm, out_hbm,
              stage, g0, g1, d0, d1, gsem0, gsem1, wsem0, wsem1):
    bj = lax.axis_index("s") * NC + lax.axis_index("c")
    gbuf, dbuf = (g0, g1), (d0, d1)
    gsem, wsem = (gsem0, gsem1), (wsem0, wsem1)

    pltpu.sync_copy(x4_hbm.at[:, bj], stage)

    lane = lax.iota(jnp.int32, LANES)
    gsel = [(k * LANES + lane) >> 3 for k in range(D_MODEL // LANES)]
    rsel = [(k * LANES + lane) & 7 for k in range(D_MODEL // LANES)]

    def idx_slice(s):
        return stage.at[s >> 3, pl.ds((s & 7) * 128, CHUNK)]

    def start_gather(s, b):
        pltpu.async_copy(table_hbm.at[idx_slice(s)], gbuf[b], gsem[b])

    def wait_gather(b):
        pltpu.make_async_copy(
            table_hbm.at[idx_slice(0)], gbuf[b], gsem[b]).wait()

    def start_write(s, b):
        pltpu.async_copy(dbuf[b].at[:, :, :, pl.ds(0, 128)],
                         out_hbm.at[pl.ds(s, CS), :, bj], wsem[b])

    def wait_write(b):
        pltpu.make_async_copy(dbuf[b].at[:, :, :, pl.ds(0, 128)],
                              out_hbm.at[pl.ds(0, CS), :, bj], wsem[b]).wait()

    start_gather(0, 0)
    start_gather(CS, 1)

    def do_pair(step, carry):
        for b in (0, 1):
            s = (step * 2 + b) * CS
            wait_gather(b)

            @pl.when(s >= 2 * CS)
            def _():
                wait_write(b)

            for sl in range(CS):
                slv = jnp.full((LANES,), sl, jnp.int32)

                def trans_row(bp, c):
                    cv = jnp.full((LANES,), bp, jnp.int32)
                    for k in range(D_MODEL // LANES):
                        vals = gbuf[b][sl * 128 + bp, pl.ds(k * LANES, LANES)]
                        plsc.store_scatter(
                            dbuf[b], [slv, gsel[k], rsel[k], cv],
                            vals * SCALE)
                    return c

                lax.fori_loop(0, 128, trans_row, 0, unroll=4)

            start_write(s, b)

            @pl.when(s + 2 * CS < S)
            def _():
                start_gather(s + 2 * CS, b)
        return carry

    lax.fori_loop(0, S // (2 * CS), do_pair, 0)
    wait_write(0)
    wait_write(1)


def kernel(x, lut):
    x4 = (x.astype(jnp.int32).reshape(NBJ, 128, NSI, 8)
          .transpose(2, 0, 3, 1).reshape(NSI, NBJ, 1024))

    out5 = pl.kernel(
        _emb_body,
        out_type=jax.ShapeDtypeStruct((S, 8, NBJ, 8, 128), jnp.float32),
        mesh=plsc.VectorSubcoreMesh(core_axis_name="c", subcore_axis_name="s"),
        compiler_params=pltpu.CompilerParams(
            use_tc_tiling_on_sc=False, needs_layout_passes=False),
        scratch_types=[
            pltpu.VMEM((NSI, 1024), jnp.int32),
            pltpu.VMEM((CHUNK, D_MODEL), jnp.float32),
            pltpu.VMEM((CHUNK, D_MODEL), jnp.float32),
            pltpu.VMEM((CS, 8, 8, PITCH), jnp.float32),
            pltpu.VMEM((CS, 8, 8, PITCH), jnp.float32),
            pltpu.SemaphoreType.DMA,
            pltpu.SemaphoreType.DMA,
            pltpu.SemaphoreType.DMA,
            pltpu.SemaphoreType.DMA,
        ],
    )(lut, x4)

    o = (out5.transpose(2, 4, 0, 1, 3)
         .reshape(B, S, D_MODEL))
    return o
